# baseline (speedup 1.0000x reference)
import functools

import jax
import jax.numpy as jnp
from jax import lax
from jax.experimental import pallas as pl
from jax.experimental.pallas import tpu as pltpu
from jax.experimental.pallas import tpu_sc as plsc

VOCAB = 100000
EMBED_DIM = 64
MAX_PACKED = 4
NUM_TOKENS = 16384

_INFO = plsc.get_sparse_core_info()
_NC = _INFO.num_cores
_NS = _INFO.num_subcores
_NW = _NC * _NS
_NROWS = MAX_PACKED * EMBED_DIM
_NBLK_FULL = VOCAB // 128
_TAIL = VOCAB - _NBLK_FULL * 128
_TAIL_WID = _NBLK_FULL % _NW
_ACHUNK = 2048
_TOKM = NUM_TOKENS - 1


def _gather_body(x_hbm, base_hbm, tt_hbm, out_hbm,
                 xc, bc, e_l, blockbuf2, tailbuf,
                 acc_e, outstage2, scatidx, bsem, ssem):
    wid = lax.axis_index("s") * _NC + lax.axis_index("c")
    iota = lax.iota(jnp.int32, 16)

    pltpu.async_copy(
        tt_hbm.at[:, pl.ds(pl.multiple_of(wid * 128, 128), 128)],
        blockbuf2.at[0],
        bsem,
    )

    widv = jnp.broadcast_to(wid, (16,))

    def pack_and_mask(i, q):
        v = xc[pl.ds(i * 16, 16)]
        s = bc[pl.ds(i * 16, 16)]
        mine = (lax.shift_right_logical(v, 7) % _NW) == widv
        tok = iota + jnp.broadcast_to(q * _ACHUNK + i * 16, (16,))
        e = (lax.shift_left(lax.shift_right_logical(v, 12), 23)
             | lax.shift_left(v & 127, 16)
             | lax.shift_left(s, 14)
             | tok)
        return e, mine, plsc.cumsum(mine.astype(jnp.int32))

    def chunk(q, cnt):
        pltpu.sync_copy(x_hbm.at[pl.ds(q * _ACHUNK, _ACHUNK)], xc)
        pltpu.sync_copy(base_hbm.at[pl.ds(q * _ACHUNK, _ACHUNK)], bc)

        def scan(h, cnt2):
            e1, m1, p1 = pack_and_mask(h * 2, q)
            e2, m2, p2 = pack_and_mask(h * 2 + 1, q)
            c16 = jnp.broadcast_to(cnt2, (16,))
            plsc.store_scatter(e_l, [c16 + p1 - 1], e1, mask=m1)
            n1 = p1[15]
            plsc.store_scatter(
                e_l, [c16 + jnp.broadcast_to(n1, (16,)) + p2 - 1], e2, mask=m2
            )
            return cnt2 + n1 + p2[15]

        return lax.fori_loop(0, _ACHUNK // 32, scan, cnt)

    cnt = lax.fori_loop(0, NUM_TOKENS // _ACHUNK, chunk, jnp.int32(0))
    nvregs = (cnt + 15) // 16
    cntv = jnp.broadcast_to(cnt, (16,))

    def emit_chunk(nvalid, bufref, ec):
        ob = ec & 1

        @pl.when(ec >= 2)
        def _():
            pltpu.make_async_copy(
                out_hbm.at[pl.ds(0, 16)], outstage2.at[0], ssem
            ).wait()

        lanemask = iota < jnp.broadcast_to(nvalid, (16,))
        ev = acc_e[pl.ds(0, 16)]
        e_last = acc_e[pl.ds(nvalid - 1, 16)][0]
        ec_v = jnp.where(lanemask, ev, jnp.broadcast_to(e_last, (16,)))
        idxv = ec_v & _TOKM
        obvec = jnp.broadcast_to(ob, (16,))
        plsc.store_scatter(scatidx, [obvec, iota], idxv)
        lane_vec = lax.shift_right_logical(ec_v, 16) & 127
        row0_vec = lax.shift_left(lax.shift_right_logical(ec_v, 14) & 3, 6)
        for c in range(EMBED_DIM):
            vals = plsc.load_gather(bufref, [row0_vec + c, lane_vec])
            plsc.store_scatter(
                outstage2,
                [obvec, iota, jnp.broadcast_to(jnp.int32(c), (16,))],
                vals,
            )

        @pl.when(ob == 0)
        def _():
            pltpu.async_copy(
                outstage2.at[0], out_hbm.at[scatidx.at[0]], ssem
            )

        @pl.when(ob == 1)
        def _():
            pltpu.async_copy(
                outstage2.at[1], out_hbm.at[scatidx.at[1]], ssem
            )

        return ec + 1

    def process_block(jblk, bufref, ec):
        jv = jnp.broadcast_to(jblk, (16,))

        def append(i, acnt):
            valid = (jnp.broadcast_to(i * 16, (16,)) + iota) < cntv
            e = e_l[pl.ds(i * 16, 16)]
            m = valid & (lax.shift_right_logical(e, 23) == jv)
            pref = plsc.cumsum(m.astype(jnp.int32))
            plsc.store_scatter(
                acc_e, [jnp.broadcast_to(acnt, (16,)) + pref - 1], e, mask=m
            )
            return acnt + pref[15]

        def flush_one(carry):
            a, e = carry
            e = emit_chunk(jnp.int32(16), bufref, e)
            t0 = acc_e[pl.ds(16, 16)]
            t1 = acc_e[pl.ds(32, 16)]
            acc_e[pl.ds(0, 16)] = t0
            acc_e[pl.ds(16, 16)] = t1
            return (a - 16, e)

        def scan2(h, carry):
            acnt, ec2 = carry
            acnt = append(h * 2, acnt)
            acnt = append(h * 2 + 1, acnt)
            carry = (acnt, ec2)
            carry = lax.cond(carry[0] >= 16, flush_one, lambda c: c, carry)
            carry = lax.cond(carry[0] >= 16, flush_one, lambda c: c, carry)
            return carry

        acnt, ec2 = lax.fori_loop(
            0, (nvregs + 1) // 2, scan2, (jnp.int32(0), ec)
        )

        def final(e):
            return emit_chunk(acnt, bufref, e)

        return lax.cond(acnt > 0, final, lambda e: e, ec2)

    nfull = jnp.where(wid <= (_NBLK_FULL - 1) % _NW,
                      1 + (_NBLK_FULL - 1) // _NW,
                      (_NBLK_FULL + _NW - 1 - wid) // _NW)

    def fire_block(j, b):
        off = pl.multiple_of((wid + _NW * j) * 128, 128)
        pltpu.async_copy(
            tt_hbm.at[:, pl.ds(off, 128)], blockbuf2.at[b], bsem
        )

    def blk_loop(j2, ecnt):
        for b in range(2):
            j = j2 * 2 + b

            def go(e, j=j, b=b):
                pltpu.make_async_copy(
                    tt_hbm.at[:, pl.ds(0, 128)], blockbuf2.at[b], bsem
                ).wait()

                @pl.when(j + 1 < nfull)
                def _():
                    fire_block(j + 1, 1 - b)

                return process_block(j, blockbuf2.at[b], e)

            ecnt = lax.cond(j < nfull, go, lambda e: e, ecnt)

        return ecnt

    ecnt = lax.fori_loop(0, (nfull + 1) // 2, blk_loop, jnp.int32(0))

    def tail(ec):
        def stage_tail(q, c):
            r0 = q * 64
            pltpu.sync_copy(
                tt_hbm.at[pl.ds(r0, 64), pl.ds(_NBLK_FULL * 128, _TAIL)],
                tailbuf,
            )

            def cp(r, c2):
                blockbuf2[0, r0 + r, pl.ds(0, 16)] = tailbuf[r, pl.ds(0, 16)]
                blockbuf2[0, r0 + r, pl.ds(16, 16)] = tailbuf[r, pl.ds(16, 16)]
                return c2

            lax.fori_loop(0, 64, cp, jnp.int32(0))
            return c

        lax.fori_loop(0, _NROWS // 64, stage_tail, jnp.int32(0))
        return process_block(jnp.int32(_NBLK_FULL // _NW), blockbuf2.at[0], ec)

    ecnt = lax.cond(wid == _TAIL_WID, tail, lambda e: e, ecnt)

    @pl.when(ecnt >= 1)
    def _():
        pltpu.make_async_copy(
            out_hbm.at[pl.ds(0, 16)], outstage2.at[0], ssem
        ).wait()

    @pl.when(ecnt >= 2)
    def _():
        pltpu.make_async_copy(
            out_hbm.at[pl.ds(0, 16)], outstage2.at[0], ssem
        ).wait()


@jax.jit
def _embedding_gather(x, base_indices, tt2d):
    mesh = plsc.VectorSubcoreMesh(core_axis_name="c", subcore_axis_name="s")
    kern = functools.partial(
        pl.kernel,
        mesh=mesh,
        compiler_params=pltpu.CompilerParams(needs_layout_passes=False),
        out_type=jax.ShapeDtypeStruct((NUM_TOKENS, 128), jnp.float32),
        scratch_types=[
            pltpu.VMEM((_ACHUNK,), jnp.int32),
            pltpu.VMEM((_ACHUNK,), jnp.int32),
            pltpu.VMEM((NUM_TOKENS + 32,), jnp.int32),
            pltpu.VMEM((2, _NROWS, 128), jnp.float32),
            pltpu.VMEM((64, _TAIL), jnp.float32),
            pltpu.VMEM((48,), jnp.int32),
            pltpu.VMEM((2, 16, 128), jnp.float32),
            pltpu.VMEM((2, 16), jnp.int32),
            pltpu.SemaphoreType.DMA,
            pltpu.SemaphoreType.DMA,
        ],
    )(_gather_body)
    return kern(x, base_indices, tt2d)


def kernel(x, base_indices, packed_weights):
    tt2d = jnp.transpose(packed_weights, (0, 2, 1)).reshape(_NROWS, VOCAB)
    out = _embedding_gather(
        x.astype(jnp.int32), base_indices.astype(jnp.int32), tt2d
    )
    return out[:, :EMBED_DIM]


# speedup vs baseline: 1.7686x; 1.7686x over previous
import functools

import jax
import jax.numpy as jnp
from jax import lax
from jax.experimental import pallas as pl
from jax.experimental.pallas import tpu as pltpu
from jax.experimental.pallas import tpu_sc as plsc

VOCAB = 100000
EMBED_DIM = 64
MAX_PACKED = 4
NUM_TOKENS = 16384

_INFO = plsc.get_sparse_core_info()
_NC = _INFO.num_cores
_NS = _INFO.num_subcores
_NW = _NC * _NS
_NROWS = MAX_PACKED * EMBED_DIM
_NBLK_FULL = VOCAB // 128
_TAIL = VOCAB - _NBLK_FULL * 128
_TAIL_WID = _NBLK_FULL % _NW
_VMASK = (1 << 17) - 1
_ACHUNK = 2048


def _gather_body(x_hbm, base_hbm, tt_hbm, out_hbm,
                 xc, bc, tok_l, flat_l, blockbuf2, tailbuf,
                 acc_tok, acc_flat, outstage2, scatidx, bsem, ssem):
    wid = lax.axis_index("s") * _NC + lax.axis_index("c")
    iota = lax.iota(jnp.int32, 16)

    pltpu.async_copy(
        tt_hbm.at[:, pl.ds(pl.multiple_of(wid * 128, 128), 128)],
        blockbuf2.at[0],
        bsem,
    )

    def chunk(q, cnt):
        pltpu.sync_copy(x_hbm.at[pl.ds(q * _ACHUNK, _ACHUNK)], xc)
        pltpu.sync_copy(base_hbm.at[pl.ds(q * _ACHUNK, _ACHUNK)], bc)

        def scan(i, cnt2):
            v = xc[pl.ds(i * 16, 16)]
            s = bc[pl.ds(i * 16, 16)]
            blk = lax.shift_right_logical(v, 7)
            mine = (blk % _NW) == jnp.broadcast_to(wid, (16,))
            tok = iota + jnp.broadcast_to(q * _ACHUNK + i * 16, (16,))
            flat = lax.shift_left(s, 17) | v
            pref = plsc.cumsum(mine.astype(jnp.int32))
            pos = jnp.broadcast_to(cnt2, (16,)) + pref - 1
            plsc.store_scatter(tok_l, [pos], tok, mask=mine)
            plsc.store_scatter(flat_l, [pos], flat, mask=mine)
            return cnt2 + pref[15]

        return lax.fori_loop(0, _ACHUNK // 16, scan, cnt)

    cnt = lax.fori_loop(0, NUM_TOKENS // _ACHUNK, chunk, jnp.int32(0))
    nvregs = (cnt + 15) // 16

    def emit_chunk(nvalid, bufref, ec):
        ob = ec & 1

        @pl.when(ec >= 2)
        def _():
            pltpu.make_async_copy(
                out_hbm.at[pl.ds(0, 16)], outstage2.at[0], ssem
            ).wait()

        nv = jnp.broadcast_to(nvalid, (16,))
        lanemask = iota < nv
        tokv = acc_tok[pl.ds(0, 16)]
        flatv = acc_flat[pl.ds(0, 16)]
        tok_last = acc_tok[pl.ds(nvalid - 1, 16)][0]
        flat_last = acc_flat[pl.ds(nvalid - 1, 16)][0]
        idxv = jnp.where(lanemask, tokv, jnp.broadcast_to(tok_last, (16,)))
        flatc = jnp.where(lanemask, flatv, jnp.broadcast_to(flat_last, (16,)))
        obvec = jnp.broadcast_to(ob, (16,))
        plsc.store_scatter(scatidx, [obvec, iota], idxv)
        vvec = flatc & _VMASK
        lane_vec = vvec & 127
        row0_vec = lax.shift_left(lax.shift_right_logical(flatc, 17), 6)

        def extract(c, carry):
            cv = jnp.broadcast_to(c, (16,))
            vals = plsc.load_gather(bufref, [row0_vec + cv, lane_vec])
            plsc.store_scatter(outstage2, [obvec, iota, cv], vals)
            return carry

        lax.fori_loop(0, EMBED_DIM, extract, jnp.int32(0))

        @pl.when(ob == 0)
        def _():
            pltpu.async_copy(
                outstage2.at[0], out_hbm.at[scatidx.at[0]], ssem
            )

        @pl.when(ob == 1)
        def _():
            pltpu.async_copy(
                outstage2.at[1], out_hbm.at[scatidx.at[1]], ssem
            )

        return ec + 1

    def process_block(B, bufref, ec):
        def scan2(i, carry):
            acnt, ec = carry
            valid = (jnp.broadcast_to(i * 16, (16,)) + iota) \
                < jnp.broadcast_to(cnt, (16,))
            fl = flat_l[pl.ds(i * 16, 16)]
            tk = tok_l[pl.ds(i * 16, 16)]
            m = valid & (lax.shift_right_logical(fl & _VMASK, 7)
                         == jnp.broadcast_to(B, (16,)))
            pref = plsc.cumsum(m.astype(jnp.int32))
            pos = jnp.broadcast_to(acnt, (16,)) + pref - 1
            plsc.store_scatter(acc_tok, [pos], tk, mask=m)
            plsc.store_scatter(acc_flat, [pos], fl, mask=m)
            acnt = acnt + pref[15]

            def flush(carry):
                a, e = carry
                e = emit_chunk(jnp.int32(16), bufref, e)
                t = acc_tok[pl.ds(16, 16)]
                f = acc_flat[pl.ds(16, 16)]
                acc_tok[pl.ds(0, 16)] = t
                acc_flat[pl.ds(0, 16)] = f
                return (a - 16, e)

            return lax.cond(acnt >= 16, flush, lambda carry: carry, (acnt, ec))

        acnt, ec2 = lax.fori_loop(0, nvregs, scan2, (jnp.int32(0), ec))

        def final(e):
            return emit_chunk(acnt, bufref, e)

        return lax.cond(acnt > 0, final, lambda e: e, ec2)

    nfull = jnp.where(wid <= (_NBLK_FULL - 1) % _NW,
                      1 + (_NBLK_FULL - 1) // _NW,
                      (_NBLK_FULL + _NW - 1 - wid) // _NW)

    def fire_block(j, b):
        B = wid + _NW * j
        off = pl.multiple_of(B * 128, 128)
        pltpu.async_copy(
            tt_hbm.at[:, pl.ds(off, 128)], blockbuf2.at[b], bsem
        )

    def blk_loop(j2, ecnt):
        for b in range(2):
            j = j2 * 2 + b

            def go(e, j=j, b=b):
                pltpu.make_async_copy(
                    tt_hbm.at[:, pl.ds(0, 128)], blockbuf2.at[b], bsem
                ).wait()

                @pl.when(j + 1 < nfull)
                def _():
                    fire_block(j + 1, 1 - b)

                return process_block(wid + _NW * j, blockbuf2.at[b], e)

            ecnt = lax.cond(j < nfull, go, lambda e: e, ecnt)

        return ecnt

    ecnt = lax.fori_loop(0, (nfull + 1) // 2, blk_loop, jnp.int32(0))

    def tail(ec):
        def stage_tail(q, c):
            r0 = q * 64
            pltpu.sync_copy(
                tt_hbm.at[pl.ds(r0, 64), pl.ds(_NBLK_FULL * 128, _TAIL)],
                tailbuf,
            )

            def cp(r, c2):
                blockbuf2[0, r0 + r, pl.ds(0, 16)] = tailbuf[r, pl.ds(0, 16)]
                blockbuf2[0, r0 + r, pl.ds(16, 16)] = tailbuf[r, pl.ds(16, 16)]
                return c2

            lax.fori_loop(0, 64, cp, jnp.int32(0))
            return c

        lax.fori_loop(0, _NROWS // 64, stage_tail, jnp.int32(0))
        return process_block(jnp.int32(_NBLK_FULL), blockbuf2.at[0], ec)

    ecnt = lax.cond(wid == _TAIL_WID, tail, lambda e: e, ecnt)

    @pl.when(ecnt >= 1)
    def _():
        pltpu.make_async_copy(
            out_hbm.at[pl.ds(0, 16)], outstage2.at[0], ssem
        ).wait()

    @pl.when(ecnt >= 2)
    def _():
        pltpu.make_async_copy(
            out_hbm.at[pl.ds(0, 16)], outstage2.at[0], ssem
        ).wait()


@jax.jit
def _embedding_gather(x, base_indices, tt2d):
    mesh = plsc.VectorSubcoreMesh(core_axis_name="c", subcore_axis_name="s")
    kern = functools.partial(
        pl.kernel,
        mesh=mesh,
        compiler_params=pltpu.CompilerParams(needs_layout_passes=False),
        out_type=jax.ShapeDtypeStruct((NUM_TOKENS, 128), jnp.float32),
        scratch_types=[
            pltpu.VMEM((_ACHUNK,), jnp.int32),
            pltpu.VMEM((_ACHUNK,), jnp.int32),
            pltpu.VMEM((NUM_TOKENS + 16,), jnp.int32),
            pltpu.VMEM((NUM_TOKENS + 16,), jnp.int32),
            pltpu.VMEM((2, _NROWS, 128), jnp.float32),
            pltpu.VMEM((64, _TAIL), jnp.float32),
            pltpu.VMEM((48,), jnp.int32),
            pltpu.VMEM((48,), jnp.int32),
            pltpu.VMEM((2, 16, 128), jnp.float32),
            pltpu.VMEM((2, 16), jnp.int32),
            pltpu.SemaphoreType.DMA,
            pltpu.SemaphoreType.DMA,
        ],
    )(_gather_body)
    return kern(x, base_indices, tt2d)


def kernel(x, base_indices, packed_weights):
    tt2d = jnp.transpose(packed_weights, (0, 2, 1)).reshape(_NROWS, VOCAB)
    out = _embedding_gather(
        x.astype(jnp.int32), base_indices.astype(jnp.int32), tt2d
    )
    return out[:, :EMBED_DIM]


# speedup vs baseline: 1.7767x; 1.0046x over previous
import functools

import jax
import jax.numpy as jnp
from jax import lax
from jax.experimental import pallas as pl
from jax.experimental.pallas import tpu as pltpu
from jax.experimental.pallas import tpu_sc as plsc

VOCAB = 100000
EMBED_DIM = 64
MAX_PACKED = 4
NUM_TOKENS = 16384

_INFO = plsc.get_sparse_core_info()
_NC = _INFO.num_cores
_NS = _INFO.num_subcores
_NW = _NC * _NS
_NROWS = MAX_PACKED * EMBED_DIM
_NBLK_FULL = VOCAB // 128
_TAIL = VOCAB - _NBLK_FULL * 128
_TAIL_WID = _NBLK_FULL % _NW
_VMASK = (1 << 17) - 1
_ACHUNK = 2048


def _gather_body(x_hbm, base_hbm, tt_hbm, out_hbm,
                 xc, bc, tok_l, flat_l, blockbuf2, tailbuf,
                 acc_tok, acc_flat, outstage2, scatidx, bsem, ssem):
    wid = lax.axis_index("s") * _NC + lax.axis_index("c")
    iota = lax.iota(jnp.int32, 16)

    pltpu.async_copy(
        tt_hbm.at[:, pl.ds(pl.multiple_of(wid * 128, 128), 128)],
        blockbuf2.at[0],
        bsem,
    )

    def chunk(q, cnt):
        pltpu.sync_copy(x_hbm.at[pl.ds(q * _ACHUNK, _ACHUNK)], xc)
        pltpu.sync_copy(base_hbm.at[pl.ds(q * _ACHUNK, _ACHUNK)], bc)

        def scan(i, cnt2):
            v = xc[pl.ds(i * 16, 16)]
            s = bc[pl.ds(i * 16, 16)]
            blk = lax.shift_right_logical(v, 7)
            mine = (blk % _NW) == jnp.broadcast_to(wid, (16,))
            tok = iota + jnp.broadcast_to(q * _ACHUNK + i * 16, (16,))
            flat = lax.shift_left(s, 17) | v
            pref = plsc.cumsum(mine.astype(jnp.int32))
            pos = jnp.broadcast_to(cnt2, (16,)) + pref - 1
            plsc.store_scatter(tok_l, [pos], tok, mask=mine)
            plsc.store_scatter(flat_l, [pos], flat, mask=mine)
            return cnt2 + pref[15]

        return lax.fori_loop(0, _ACHUNK // 16, scan, cnt)

    cnt = lax.fori_loop(0, NUM_TOKENS // _ACHUNK, chunk, jnp.int32(0))
    nvregs = (cnt + 15) // 16

    def emit_chunk(nvalid, bsel, ec):
        ob = ec & 1

        @pl.when(ec >= 2)
        def _():
            pltpu.make_async_copy(
                out_hbm.at[pl.ds(0, 16)], outstage2.at[0], ssem
            ).wait()

        nv = jnp.broadcast_to(nvalid, (16,))
        lanemask = iota < nv
        tokv = acc_tok[pl.ds(0, 16)]
        flatv = acc_flat[pl.ds(0, 16)]
        tok_last = acc_tok[pl.ds(nvalid - 1, 16)][0]
        flat_last = acc_flat[pl.ds(nvalid - 1, 16)][0]
        idxv = jnp.where(lanemask, tokv, jnp.broadcast_to(tok_last, (16,)))
        flatc = jnp.where(lanemask, flatv, jnp.broadcast_to(flat_last, (16,)))
        obvec = jnp.broadcast_to(ob, (16,))
        plsc.store_scatter(scatidx, [obvec, iota], idxv)
        vvec = flatc & _VMASK
        lane_vec = vvec & 127
        row0_vec = lax.shift_left(lax.shift_right_logical(flatc, 17), 6)

        bselv = jnp.broadcast_to(bsel, (16,))

        def extract(c, carry):
            cv = jnp.broadcast_to(c, (16,))
            vals = plsc.load_gather(blockbuf2, [bselv, row0_vec + cv, lane_vec])
            plsc.store_scatter(outstage2, [obvec, iota, cv], vals)
            return carry

        lax.fori_loop(0, EMBED_DIM, extract, jnp.int32(0))

        @pl.when(ob == 0)
        def _():
            pltpu.async_copy(
                outstage2.at[0], out_hbm.at[scatidx.at[0]], ssem
            )

        @pl.when(ob == 1)
        def _():
            pltpu.async_copy(
                outstage2.at[1], out_hbm.at[scatidx.at[1]], ssem
            )

        return ec + 1

    def process_block(B, bsel, ec):
        def scan2(i, carry):
            acnt, ec = carry
            valid = (jnp.broadcast_to(i * 16, (16,)) + iota) \
                < jnp.broadcast_to(cnt, (16,))
            fl = flat_l[pl.ds(i * 16, 16)]
            tk = tok_l[pl.ds(i * 16, 16)]
            m = valid & (lax.shift_right_logical(fl & _VMASK, 7)
                         == jnp.broadcast_to(B, (16,)))
            pref = plsc.cumsum(m.astype(jnp.int32))
            pos = jnp.broadcast_to(acnt, (16,)) + pref - 1
            plsc.store_scatter(acc_tok, [pos], tk, mask=m)
            plsc.store_scatter(acc_flat, [pos], fl, mask=m)
            acnt = acnt + pref[15]

            def flush(carry):
                a, e = carry
                e = emit_chunk(jnp.int32(16), bsel, e)
                t = acc_tok[pl.ds(16, 16)]
                f = acc_flat[pl.ds(16, 16)]
                acc_tok[pl.ds(0, 16)] = t
                acc_flat[pl.ds(0, 16)] = f
                return (a - 16, e)

            return lax.cond(acnt >= 16, flush, lambda carry: carry, (acnt, ec))

        acnt, ec2 = lax.fori_loop(0, nvregs, scan2, (jnp.int32(0), ec))

        def final(e):
            return emit_chunk(acnt, bsel, e)

        return lax.cond(acnt > 0, final, lambda e: e, ec2)

    nfull = jnp.where(wid <= (_NBLK_FULL - 1) % _NW,
                      1 + (_NBLK_FULL - 1) // _NW,
                      (_NBLK_FULL + _NW - 1 - wid) // _NW)

    def fire_block(j, b):
        B = wid + _NW * j
        off = pl.multiple_of(B * 128, 128)
        pltpu.async_copy(
            tt_hbm.at[:, pl.ds(off, 128)], blockbuf2.at[b], bsem
        )

    def blk_loop(j, ecnt):
        pltpu.make_async_copy(
            tt_hbm.at[:, pl.ds(0, 128)], blockbuf2.at[0], bsem
        ).wait()
        b = j & 1

        @pl.when((j + 1 < nfull) & (b == 0))
        def _():
            fire_block(j + 1, 1)

        @pl.when((j + 1 < nfull) & (b == 1))
        def _():
            fire_block(j + 1, 0)

        return process_block(wid + _NW * j, b, ecnt)

    ecnt = lax.fori_loop(0, nfull, blk_loop, jnp.int32(0))

    def tail(ec):
        def stage_tail(q, c):
            r0 = q * 64
            pltpu.sync_copy(
                tt_hbm.at[pl.ds(r0, 64), pl.ds(_NBLK_FULL * 128, _TAIL)],
                tailbuf,
            )

            def cp(r, c2):
                blockbuf2[0, r0 + r, pl.ds(0, 16)] = tailbuf[r, pl.ds(0, 16)]
                blockbuf2[0, r0 + r, pl.ds(16, 16)] = tailbuf[r, pl.ds(16, 16)]
                return c2

            lax.fori_loop(0, 64, cp, jnp.int32(0))
            return c

        lax.fori_loop(0, _NROWS // 64, stage_tail, jnp.int32(0))
        return process_block(jnp.int32(_NBLK_FULL), jnp.int32(0), ec)

    ecnt = lax.cond(wid == _TAIL_WID, tail, lambda e: e, ecnt)

    @pl.when(ecnt >= 1)
    def _():
        pltpu.make_async_copy(
            out_hbm.at[pl.ds(0, 16)], outstage2.at[0], ssem
        ).wait()

    @pl.when(ecnt >= 2)
    def _():
        pltpu.make_async_copy(
            out_hbm.at[pl.ds(0, 16)], outstage2.at[0], ssem
        ).wait()


@jax.jit
def _embedding_gather(x, base_indices, tt2d):
    mesh = plsc.VectorSubcoreMesh(core_axis_name="c", subcore_axis_name="s")
    kern = functools.partial(
        pl.kernel,
        mesh=mesh,
        compiler_params=pltpu.CompilerParams(needs_layout_passes=False),
        out_type=jax.ShapeDtypeStruct((NUM_TOKENS, 128), jnp.float32),
        scratch_types=[
            pltpu.VMEM((_ACHUNK,), jnp.int32),
            pltpu.VMEM((_ACHUNK,), jnp.int32),
            pltpu.VMEM((NUM_TOKENS + 16,), jnp.int32),
            pltpu.VMEM((NUM_TOKENS + 16,), jnp.int32),
            pltpu.VMEM((2, _NROWS, 128), jnp.float32),
            pltpu.VMEM((64, _TAIL), jnp.float32),
            pltpu.VMEM((48,), jnp.int32),
            pltpu.VMEM((48,), jnp.int32),
            pltpu.VMEM((2, 16, 128), jnp.float32),
            pltpu.VMEM((2, 16), jnp.int32),
            pltpu.SemaphoreType.DMA,
            pltpu.SemaphoreType.DMA,
        ],
    )(_gather_body)
    return kern(x, base_indices, tt2d)


def kernel(x, base_indices, packed_weights):
    tt2d = jnp.transpose(packed_weights, (0, 2, 1)).reshape(_NROWS, VOCAB)
    out = _embedding_gather(
        x.astype(jnp.int32), base_indices.astype(jnp.int32), tt2d
    )
    return out[:, :EMBED_DIM]


# speedup vs baseline: 1.8358x; 1.0333x over previous
import functools

import jax
import jax.numpy as jnp
from jax import lax
from jax.experimental import pallas as pl
from jax.experimental.pallas import tpu as pltpu
from jax.experimental.pallas import tpu_sc as plsc

VOCAB = 100000
EMBED_DIM = 64
MAX_PACKED = 4
NUM_TOKENS = 16384

_INFO = plsc.get_sparse_core_info()
_NC = _INFO.num_cores
_NS = _INFO.num_subcores
_NW = _NC * _NS
_NROWS = MAX_PACKED * EMBED_DIM
_NBLK_FULL = VOCAB // 128
_TAIL = VOCAB - _NBLK_FULL * 128
_TAIL_WID = _NBLK_FULL % _NW
_VMASK = (1 << 17) - 1
_ACHUNK = 4096


def _gather_body(x_hbm, base_hbm, tt_hbm, out_hbm,
                 xc, bc, tok_l, flat_l, blockbuf2, tailbuf,
                 acc_tok, acc_flat, outstage2, scatidx, bsem, ssem):
    wid = lax.axis_index("s") * _NC + lax.axis_index("c")
    iota = lax.iota(jnp.int32, 16)

    pltpu.async_copy(
        tt_hbm.at[:, pl.ds(pl.multiple_of(wid * 128, 128), 128)],
        blockbuf2.at[0],
        bsem,
    )

    def chunk(q, cnt):
        pltpu.sync_copy(x_hbm.at[pl.ds(q * _ACHUNK, _ACHUNK)], xc)
        pltpu.sync_copy(base_hbm.at[pl.ds(q * _ACHUNK, _ACHUNK)], bc)

        def scan(i, cnt2):
            v = xc[pl.ds(i * 16, 16)]
            s = bc[pl.ds(i * 16, 16)]
            blk = lax.shift_right_logical(v, 7)
            mine = (blk % _NW) == jnp.broadcast_to(wid, (16,))
            tok = iota + jnp.broadcast_to(q * _ACHUNK + i * 16, (16,))
            flat = lax.shift_left(s, 17) | v
            pref = plsc.cumsum(mine.astype(jnp.int32))
            pos = jnp.broadcast_to(cnt2, (16,)) + pref - 1
            plsc.store_scatter(tok_l, [pos], tok, mask=mine)
            plsc.store_scatter(flat_l, [pos], flat, mask=mine)
            return cnt2 + pref[15]

        return lax.fori_loop(0, _ACHUNK // 16, scan, cnt)

    cnt = lax.fori_loop(0, NUM_TOKENS // _ACHUNK, chunk, jnp.int32(0))
    nvregs = (cnt + 15) // 16

    def emit_chunk(nvalid, bsel, ec):
        ob = ec & 1

        @pl.when(ec >= 2)
        def _():
            pltpu.make_async_copy(
                out_hbm.at[pl.ds(0, 16)], outstage2.at[0], ssem
            ).wait()

        nv = jnp.broadcast_to(nvalid, (16,))
        lanemask = iota < nv
        tokv = acc_tok[pl.ds(0, 16)]
        flatv = acc_flat[pl.ds(0, 16)]
        tok_last = acc_tok[pl.ds(nvalid - 1, 16)][0]
        flat_last = acc_flat[pl.ds(nvalid - 1, 16)][0]
        idxv = jnp.where(lanemask, tokv, jnp.broadcast_to(tok_last, (16,)))
        flatc = jnp.where(lanemask, flatv, jnp.broadcast_to(flat_last, (16,)))
        obvec = jnp.broadcast_to(ob, (16,))
        plsc.store_scatter(scatidx, [obvec, iota], idxv)
        vvec = flatc & _VMASK
        lane_vec = vvec & 127
        row0_vec = lax.shift_left(lax.shift_right_logical(flatc, 17), 6)

        bselv = jnp.broadcast_to(bsel, (16,))

        def extract(c2, carry):
            for d in range(2):
                cv = jnp.broadcast_to(c2 * 2 + d, (16,))
                vals = plsc.load_gather(
                    blockbuf2, [bselv, row0_vec + cv, lane_vec]
                )
                plsc.store_scatter(outstage2, [obvec, iota, cv], vals)
            return carry

        lax.fori_loop(0, EMBED_DIM // 2, extract, jnp.int32(0))

        @pl.when(ob == 0)
        def _():
            pltpu.async_copy(
                outstage2.at[0], out_hbm.at[scatidx.at[0]], ssem
            )

        @pl.when(ob == 1)
        def _():
            pltpu.async_copy(
                outstage2.at[1], out_hbm.at[scatidx.at[1]], ssem
            )

        return ec + 1

    def process_block(B, bsel, ec):
        def scan2(i, carry):
            acnt, ec = carry
            valid = (jnp.broadcast_to(i * 16, (16,)) + iota) \
                < jnp.broadcast_to(cnt, (16,))
            fl = flat_l[pl.ds(i * 16, 16)]
            tk = tok_l[pl.ds(i * 16, 16)]
            m = valid & (lax.shift_right_logical(fl & _VMASK, 7)
                         == jnp.broadcast_to(B, (16,)))
            pref = plsc.cumsum(m.astype(jnp.int32))
            pos = jnp.broadcast_to(acnt, (16,)) + pref - 1
            plsc.store_scatter(acc_tok, [pos], tk, mask=m)
            plsc.store_scatter(acc_flat, [pos], fl, mask=m)
            acnt = acnt + pref[15]

            def flush(carry):
                a, e = carry
                e = emit_chunk(jnp.int32(16), bsel, e)
                t = acc_tok[pl.ds(16, 16)]
                f = acc_flat[pl.ds(16, 16)]
                acc_tok[pl.ds(0, 16)] = t
                acc_flat[pl.ds(0, 16)] = f
                return (a - 16, e)

            return lax.cond(acnt >= 16, flush, lambda carry: carry, (acnt, ec))

        acnt, ec2 = lax.fori_loop(0, nvregs, scan2, (jnp.int32(0), ec))

        def final(e):
            return emit_chunk(acnt, bsel, e)

        return lax.cond(acnt > 0, final, lambda e: e, ec2)

    nfull = jnp.where(wid <= (_NBLK_FULL - 1) % _NW,
                      1 + (_NBLK_FULL - 1) // _NW,
                      (_NBLK_FULL + _NW - 1 - wid) // _NW)

    def fire_block(j, b):
        B = wid + _NW * j
        off = pl.multiple_of(B * 128, 128)
        pltpu.async_copy(
            tt_hbm.at[:, pl.ds(off, 128)], blockbuf2.at[b], bsem
        )

    def blk_loop(j, ecnt):
        pltpu.make_async_copy(
            tt_hbm.at[:, pl.ds(0, 128)], blockbuf2.at[0], bsem
        ).wait()
        b = j & 1

        @pl.when((j + 1 < nfull) & (b == 0))
        def _():
            fire_block(j + 1, 1)

        @pl.when((j + 1 < nfull) & (b == 1))
        def _():
            fire_block(j + 1, 0)

        return process_block(wid + _NW * j, b, ecnt)

    ecnt = lax.fori_loop(0, nfull, blk_loop, jnp.int32(0))

    def tail(ec):
        def stage_tail(q, c):
            r0 = q * 64
            pltpu.sync_copy(
                tt_hbm.at[pl.ds(r0, 64), pl.ds(_NBLK_FULL * 128, _TAIL)],
                tailbuf,
            )

            def cp(r, c2):
                blockbuf2[0, r0 + r, pl.ds(0, 16)] = tailbuf[r, pl.ds(0, 16)]
                blockbuf2[0, r0 + r, pl.ds(16, 16)] = tailbuf[r, pl.ds(16, 16)]
                return c2

            lax.fori_loop(0, 64, cp, jnp.int32(0))
            return c

        lax.fori_loop(0, _NROWS // 64, stage_tail, jnp.int32(0))
        return process_block(jnp.int32(_NBLK_FULL), jnp.int32(0), ec)

    ecnt = lax.cond(wid == _TAIL_WID, tail, lambda e: e, ecnt)

    @pl.when(ecnt >= 1)
    def _():
        pltpu.make_async_copy(
            out_hbm.at[pl.ds(0, 16)], outstage2.at[0], ssem
        ).wait()

    @pl.when(ecnt >= 2)
    def _():
        pltpu.make_async_copy(
            out_hbm.at[pl.ds(0, 16)], outstage2.at[0], ssem
        ).wait()


@jax.jit
def _embedding_gather(x, base_indices, tt2d):
    mesh = plsc.VectorSubcoreMesh(core_axis_name="c", subcore_axis_name="s")
    kern = functools.partial(
        pl.kernel,
        mesh=mesh,
        compiler_params=pltpu.CompilerParams(needs_layout_passes=False),
        out_type=jax.ShapeDtypeStruct((NUM_TOKENS, 128), jnp.float32),
        scratch_types=[
            pltpu.VMEM((_ACHUNK,), jnp.int32),
            pltpu.VMEM((_ACHUNK,), jnp.int32),
            pltpu.VMEM((NUM_TOKENS + 16,), jnp.int32),
            pltpu.VMEM((NUM_TOKENS + 16,), jnp.int32),
            pltpu.VMEM((2, _NROWS, 128), jnp.float32),
            pltpu.VMEM((64, _TAIL), jnp.float32),
            pltpu.VMEM((48,), jnp.int32),
            pltpu.VMEM((48,), jnp.int32),
            pltpu.VMEM((2, 16, 128), jnp.float32),
            pltpu.VMEM((2, 16), jnp.int32),
            pltpu.SemaphoreType.DMA,
            pltpu.SemaphoreType.DMA,
        ],
    )(_gather_body)
    return kern(x, base_indices, tt2d)


def kernel(x, base_indices, packed_weights):
    tt2d = jnp.transpose(packed_weights, (0, 2, 1)).reshape(_NROWS, VOCAB)
    out = _embedding_gather(
        x.astype(jnp.int32), base_indices.astype(jnp.int32), tt2d
    )
    return out[:, :EMBED_DIM]
